# baseline (speedup 1.0000x reference)
import jax
import jax.numpy as jnp
from jax.experimental import pallas as pl
from jax.experimental.pallas import tpu as pltpu

_CHUNK_T = 512
_N_BUF = 8


def _postprocess(scores):
    e = scores.shape[-1]
    lane = jax.lax.broadcasted_iota(jnp.int32, scores.shape, 1).astype(
        jnp.float32)

    m1 = jnp.max(scores, axis=-1, keepdims=True)
    idx1 = jnp.min(jnp.where(scores == m1, lane, float(e)), axis=-1,
                   keepdims=True)
    first = lane == idx1
    s2 = jnp.where(first, -jnp.inf, scores)
    m2 = jnp.max(s2, axis=-1, keepdims=True)
    idx2 = jnp.min(jnp.where(s2 == m2, lane, float(e)), axis=-1,
                   keepdims=True)
    mask = first | (lane == idx2)

    shift = m1 + jnp.log1p(jnp.exp(m2 - m1))
    return jnp.where(mask, jnp.exp(scores - shift), jnp.float32(0.0))


def _gate_kernel(x_hbm, wt_ref, b_ref, o_ref, xbuf, sem):
    n_chunks = x_hbm.shape[0] // _CHUNK_T
    wt = wt_ref[...]
    bias = b_ref[...]

    def start_copy(c, slot):
        pltpu.make_async_copy(
            x_hbm.at[pl.ds(c * _CHUNK_T, _CHUNK_T), :],
            xbuf.at[slot],
            sem.at[slot],
        ).start()

    for i in range(_N_BUF):
        start_copy(i, i)

    for c in range(n_chunks):
        slot = c % _N_BUF
        pltpu.make_async_copy(
            x_hbm.at[pl.ds(c * _CHUNK_T, _CHUNK_T), :],
            xbuf.at[slot],
            sem.at[slot],
        ).wait()
        scores = jnp.dot(xbuf[slot], wt,
                         preferred_element_type=jnp.float32) + bias
        nxt = c + _N_BUF
        if nxt < n_chunks:
            start_copy(nxt, slot)
        o_ref[c * _CHUNK_T:(c + 1) * _CHUNK_T, :] = _postprocess(scores)


@jax.jit
def kernel(x, W, b):
    n_tokens, d_model = x.shape
    n_experts = W.shape[0]
    wt = W.T
    b2 = b.reshape(1, n_experts)
    return pl.pallas_call(
        _gate_kernel,
        in_specs=[
            pl.BlockSpec(memory_space=pl.ANY),
            pl.BlockSpec((d_model, n_experts), lambda: (0, 0)),
            pl.BlockSpec((1, n_experts), lambda: (0, 0)),
        ],
        out_specs=pl.BlockSpec((n_tokens, n_experts), lambda: (0, 0)),
        out_shape=jax.ShapeDtypeStruct((n_tokens, n_experts), jnp.float32),
        scratch_shapes=[
            pltpu.VMEM((_N_BUF, _CHUNK_T, d_model), jnp.float32),
            pltpu.SemaphoreType.DMA((_N_BUF,)),
        ],
    )(x, wt, b2)


# speedup vs baseline: 1.2274x; 1.2274x over previous
import functools

import jax
import jax.numpy as jnp
from jax.experimental import pallas as pl
from jax.experimental.pallas import tpu as pltpu

_BLOCK_T = 2048
_N_SPLIT = 4
_SUB_T = _BLOCK_T // _N_SPLIT


def _postprocess(scores):
    e = scores.shape[-1]
    lane = jax.lax.broadcasted_iota(jnp.int32, scores.shape, 1).astype(
        jnp.float32)

    m1 = jnp.max(scores, axis=-1, keepdims=True)
    idx1 = jnp.min(jnp.where(scores == m1, lane, float(e)), axis=-1,
                   keepdims=True)
    first = lane == idx1
    s2 = jnp.where(first, -jnp.inf, scores)
    m2 = jnp.max(s2, axis=-1, keepdims=True)
    idx2 = jnp.min(jnp.where(s2 == m2, lane, float(e)), axis=-1,
                   keepdims=True)
    mask = first | (lane == idx2)

    shift = m1 + jnp.log1p(jnp.exp(m2 - m1))
    return jnp.where(mask, jnp.exp(scores - shift), jnp.float32(0.0))


def _score(x, w, bias):
    return jax.lax.dot_general(
        x, w, (((1,), (1,)), ((), ())),
        preferred_element_type=jnp.float32) + bias


def _gate_kernel(*refs):
    x_refs = refs[:_N_SPLIT]
    w_ref, b_ref, o_ref = refs[_N_SPLIT:]
    w = w_ref[...]
    bias = b_ref[...]
    scores = [None] * _N_SPLIT
    scores[0] = _score(x_refs[0][...], w, bias)
    for j in range(_N_SPLIT):
        if j + 1 < _N_SPLIT:
            scores[j + 1] = _score(x_refs[j + 1][...], w, bias)
        o_ref[j * _SUB_T:(j + 1) * _SUB_T, :] = _postprocess(scores[j])
        scores[j] = None


@jax.jit
def kernel(x, W, b):
    n_tokens, d_model = x.shape
    n_experts = W.shape[0]
    b2 = b.reshape(1, n_experts)
    grid = (n_tokens // _BLOCK_T,)
    x_specs = [
        pl.BlockSpec((_SUB_T, d_model),
                     functools.partial(lambda i, j: (_N_SPLIT * i + j, 0), j=j))
        for j in range(_N_SPLIT)
    ]
    return pl.pallas_call(
        _gate_kernel,
        grid=grid,
        in_specs=x_specs + [
            pl.BlockSpec((n_experts, d_model), lambda i: (0, 0)),
            pl.BlockSpec((1, n_experts), lambda i: (0, 0)),
        ],
        out_specs=pl.BlockSpec((_BLOCK_T, n_experts), lambda i: (i, 0)),
        out_shape=jax.ShapeDtypeStruct((n_tokens, n_experts), jnp.float32),
        compiler_params=pltpu.CompilerParams(
            dimension_semantics=("arbitrary",),
        ),
    )(*([x] * _N_SPLIT), W, b2)


# speedup vs baseline: 1.2320x; 1.0037x over previous
import functools

import jax
import jax.numpy as jnp
from jax.experimental import pallas as pl
from jax.experimental.pallas import tpu as pltpu

_BLOCK_T = 2048
_N_SPLIT = 8
_SUB_T = _BLOCK_T // _N_SPLIT


def _postprocess(scores):
    e = scores.shape[-1]
    lane = jax.lax.broadcasted_iota(jnp.int32, scores.shape, 1).astype(
        jnp.float32)

    m1 = jnp.max(scores, axis=-1, keepdims=True)
    idx1 = jnp.min(jnp.where(scores == m1, lane, float(e)), axis=-1,
                   keepdims=True)
    first = lane == idx1
    s2 = jnp.where(first, -jnp.inf, scores)
    m2 = jnp.max(s2, axis=-1, keepdims=True)
    idx2 = jnp.min(jnp.where(s2 == m2, lane, float(e)), axis=-1,
                   keepdims=True)
    mask = first | (lane == idx2)

    shift = m1 + jnp.log1p(jnp.exp(m2 - m1))
    return jnp.where(mask, jnp.exp(scores - shift), jnp.float32(0.0))


def _score(x, w, bias):
    return jax.lax.dot_general(
        x, w, (((1,), (1,)), ((), ())),
        preferred_element_type=jnp.float32) + bias


def _gate_kernel(*refs):
    x_refs = refs[:_N_SPLIT]
    w_ref, b_ref, o_ref = refs[_N_SPLIT:]
    w = w_ref[...]
    bias = b_ref[...]
    scores = [None] * _N_SPLIT
    scores[0] = _score(x_refs[0][...], w, bias)
    for j in range(_N_SPLIT):
        if j + 1 < _N_SPLIT:
            scores[j + 1] = _score(x_refs[j + 1][...], w, bias)
        o_ref[j * _SUB_T:(j + 1) * _SUB_T, :] = _postprocess(scores[j])
        scores[j] = None


@jax.jit
def kernel(x, W, b):
    n_tokens, d_model = x.shape
    n_experts = W.shape[0]
    b2 = b.reshape(1, n_experts)
    grid = (n_tokens // _BLOCK_T,)
    x_specs = [
        pl.BlockSpec((_SUB_T, d_model),
                     functools.partial(lambda i, j: (_N_SPLIT * i + j, 0), j=j))
        for j in range(_N_SPLIT)
    ]
    return pl.pallas_call(
        _gate_kernel,
        grid=grid,
        in_specs=x_specs + [
            pl.BlockSpec((n_experts, d_model), lambda i: (0, 0)),
            pl.BlockSpec((1, n_experts), lambda i: (0, 0)),
        ],
        out_specs=pl.BlockSpec((_BLOCK_T, n_experts), lambda i: (i, 0)),
        out_shape=jax.ShapeDtypeStruct((n_tokens, n_experts), jnp.float32),
        compiler_params=pltpu.CompilerParams(
            dimension_semantics=("arbitrary",),
        ),
    )(*([x] * _N_SPLIT), W, b2)
